# baseline (speedup 1.0000x reference)
import jax
import jax.numpy as jnp
from jax import lax
from jax.experimental import pallas as pl
from jax.experimental.pallas import tpu as pltpu
from jax.experimental.pallas import tpu_sc as plsc

N, E, D = 10000, 320000, 128
H1, H2 = 128, 64
NC, NS, L = 2, 16, 16
NW = NC * NS
EPW = E // NW
KB = 80
NB = 126
EPW_P = NB * KB
NBUF = 3
RPT = 624
RREM = N - NS * RPT


def _sc_body(x_hbm, pki_hbm, pkf_hbm, nt_hbm, out_hbm, agg_sh,
             pb0, pb1, pb2, pf0, pf1, pf2, rb0, rb1, rb2, nd0, nd1, nd2,
             db0, db1, db2,
             es0, es1, es2, gs0, gs1, gs2, ss0, ss1, ss2):
    pbufs = (pb0, pb1, pb2)
    fbufs = (pf0, pf1, pf2)
    rbh = (rb0, rb1, rb2)
    ntds = (nd0, nd1, nd2)
    dbufs = (db0, db1, db2)
    esems = (es0, es1, es2)
    gsems = (gs0, gs1, gs2)
    ssems = (ss0, ss1, ss2)

    c = lax.axis_index("c")
    s = lax.axis_index("s")
    wid = c * NS + s
    blk0 = wid * NB

    row0 = s * RPT
    pltpu.sync_copy(x_hbm.at[pl.ds(row0, RPT)], agg_sh.at[pl.ds(row0, RPT)])

    @pl.when(s == NS - 1)
    def _init_tail():
        pltpu.sync_copy(x_hbm.at[pl.ds(NS * RPT, RREM)],
                        agg_sh.at[pl.ds(NS * RPT, RREM)])

    plsc.subcore_barrier()

    def edge_copy(b, q):
        pltpu.async_copy(pki_hbm.at[blk0 + b], pbufs[q], esems[q])
        pltpu.async_copy(pkf_hbm.at[blk0 + b], fbufs[q], esems[q])

    def wait_edge(b, q):
        pltpu.make_async_copy(pki_hbm.at[blk0 + b], pbufs[q],
                              esems[q]).wait()
        pltpu.make_async_copy(pkf_hbm.at[blk0 + b], fbufs[q],
                              esems[q]).wait()

    def fire_gathers(q):
        pltpu.async_copy(x_hbm.at[pbufs[q].at[1]], rbh[q], gsems[q])
        pltpu.async_copy(nt_hbm.at[pbufs[q].at[0]], ntds[q], gsems[q])

    def wait_gathers(q):
        pltpu.make_async_copy(x_hbm.at[pbufs[q].at[1]], rbh[q],
                              gsems[q]).wait()
        pltpu.make_async_copy(nt_hbm.at[pbufs[q].at[0]], ntds[q],
                              gsems[q]).wait()

    def fire_scatter(q):
        pltpu.async_copy(rbh[q], agg_sh.at[dbufs[q]], ssems[q], add=True)

    def wait_scatter(q):
        pltpu.make_async_copy(rbh[q], agg_sh.at[dbufs[q]], ssems[q]).wait()

    def compute(q):
        rq = rbh[q]

        def grp(g, carry):
            nt = ntds[q][pl.ds(g * L, L)]
            etv = fbufs[q][0, pl.ds(g * L, L)]
            ewv = fbufs[q][1, pl.ds(g * L, L)]
            cvec = ewv * jnp.exp(-jnp.abs(nt - etv))
            dbufs[q][pl.ds(g * L, L)] = pbufs[q][0, pl.ds(g * L, L)]
            for li in range(L):
                cf = cvec[li]
                e = g * L + li
                for j in range(D // L):
                    rq[e, pl.ds(j * L, L)] = rq[e, pl.ds(j * L, L)] * cf
            return carry

        lax.fori_loop(0, KB // L, grp, 0)

    edge_copy(0, 0)
    edge_copy(1, 1)
    wait_edge(0, 0)
    fire_gathers(0)

    def step(b, q):
        pa = (q + 1) % NBUF
        pp = (q + 2) % NBUF

        @pl.when(b >= 2)
        def _retire():
            wait_scatter(pa)

        @pl.when(b + 1 < NB)
        def _advance():
            wait_edge(b + 1, pa)
            fire_gathers(pa)

        wait_gathers(q)
        compute(q)
        fire_scatter(q)

        @pl.when(b + 2 < NB)
        def _prefetch():
            edge_copy(b + 2, pp)

    def superstep(i, carry):
        for k in range(NBUF):
            step(i * NBUF + k, k)
        return carry

    lax.fori_loop(0, NB // NBUF, superstep, 0)
    wait_scatter((NB - 2) % NBUF)
    wait_scatter((NB - 1) % NBUF)

    plsc.subcore_barrier()
    pltpu.sync_copy(agg_sh.at[pl.ds(row0, RPT)],
                    out_hbm.at[c, pl.ds(row0, RPT)])

    @pl.when(s == NS - 1)
    def _emit_tail():
        pltpu.sync_copy(agg_sh.at[pl.ds(NS * RPT, RREM)],
                        out_hbm.at[c, pl.ds(NS * RPT, RREM)])


_sc_aggregate = pl.kernel(
    _sc_body,
    out_type=jax.ShapeDtypeStruct((NC, N, D), jnp.float32),
    mesh=plsc.VectorSubcoreMesh(
        core_axis_name="c", subcore_axis_name="s", num_cores=NC,
        num_subcores=NS,
    ),
    scratch_types=(
        [pltpu.VMEM_SHARED((N, D), jnp.float32)]
        + [pltpu.VMEM((2, KB), jnp.int32) for _ in range(NBUF)]
        + [pltpu.VMEM((2, KB), jnp.float32) for _ in range(NBUF)]
        + [pltpu.VMEM((KB, D), jnp.float32) for _ in range(NBUF)]
        + [pltpu.VMEM((KB,), jnp.float32) for _ in range(NBUF)]
        + [pltpu.VMEM((KB,), jnp.int32) for _ in range(NBUF)]
        + [pltpu.SemaphoreType.DMA for _ in range(3 * NBUF)]
    ),
)


def _dense_body(p_ref, x_ref, wg_ref, w1_ref, b1_ref, g_ref, be_ref, w2_ref,
                b2_ref, o_ref):
    agg = p_ref[0] + p_ref[1] - x_ref[...]
    ne = jnp.maximum(jnp.dot(agg, wg_ref[...],
                             preferred_element_type=jnp.float32), 0.0)
    h = jnp.dot(ne, w1_ref[...], preferred_element_type=jnp.float32)
    h = jnp.maximum(h + b1_ref[...], 0.0)
    mean = jnp.mean(h, axis=0, keepdims=True)
    var = jnp.mean((h - mean) * (h - mean), axis=0, keepdims=True)
    hn = (h - mean) / jnp.sqrt(var + 1e-5) * g_ref[...] + be_ref[...]
    out = jnp.dot(hn, w2_ref[...], preferred_element_type=jnp.float32)
    o_ref[...] = jax.nn.sigmoid(out + b2_ref[...])


_dense_call = pl.pallas_call(
    _dense_body,
    out_shape=jax.ShapeDtypeStruct((N, H2), jnp.float32),
)


def _pack_edges(edge_index, edge_time, edge_weight):
    pad = EPW_P - EPW
    dst = edge_index[1].reshape(NW, EPW)
    src = edge_index[0].reshape(NW, EPW)
    et = edge_time.reshape(NW, EPW)
    ew = edge_weight.reshape(NW, EPW)
    cfg = [(0, 0), (0, pad)]
    dst, src, et, ew = [jnp.pad(a, cfg).reshape(NW, NB, KB)
                        for a in (dst, src, et, ew)]
    pki = jnp.stack([dst, src], axis=2).reshape(NW * NB, 2, KB)
    pkf = jnp.stack([et, ew], axis=2).reshape(NW * NB, 2, KB)
    return pki, pkf


def kernel(x, edge_index, edge_time, node_time, edge_weight,
           W_g, W1, b1, gamma, beta, W2, b2):
    pki, pkf = _pack_edges(edge_index, edge_time, edge_weight)
    partials = _sc_aggregate(x, pki, pkf, node_time)
    return _dense_call(partials, x, W_g, W1, b1.reshape(1, H1),
                       gamma.reshape(1, H1), beta.reshape(1, H1), W2,
                       b2.reshape(1, H2))


# speedup vs baseline: 1.0583x; 1.0583x over previous
import jax
import jax.numpy as jnp
from jax import lax
from jax.experimental import pallas as pl
from jax.experimental.pallas import tpu as pltpu
from jax.experimental.pallas import tpu_sc as plsc

N, E, D = 10000, 320000, 128
H1, H2 = 128, 64
NC, NS, L = 2, 16, 16
NW = NC * NS
EPW = E // NW
KB = 112
NB = 90
EPW_P = NB * KB
NBUF = 3
RPT = 624
RREM = N - NS * RPT


def _sc_body(x_hbm, pki_hbm, pkf_hbm, nt_hbm, out_hbm, agg_sh,
             pb0, pb1, pb2, pf0, pf1, pf2, rb0, rb1, rb2, nd0, nd1, nd2,
             db0, db1, db2,
             es0, es1, es2, gs0, gs1, gs2, ss0, ss1, ss2):
    pbufs = (pb0, pb1, pb2)
    fbufs = (pf0, pf1, pf2)
    rbh = (rb0, rb1, rb2)
    ntds = (nd0, nd1, nd2)
    dbufs = (db0, db1, db2)
    esems = (es0, es1, es2)
    gsems = (gs0, gs1, gs2)
    ssems = (ss0, ss1, ss2)

    c = lax.axis_index("c")
    s = lax.axis_index("s")
    wid = c * NS + s
    blk0 = wid * NB

    row0 = s * RPT
    pltpu.sync_copy(x_hbm.at[pl.ds(row0, RPT)], agg_sh.at[pl.ds(row0, RPT)])

    @pl.when(s == NS - 1)
    def _init_tail():
        pltpu.sync_copy(x_hbm.at[pl.ds(NS * RPT, RREM)],
                        agg_sh.at[pl.ds(NS * RPT, RREM)])

    plsc.subcore_barrier()

    def edge_copy(b, q):
        pltpu.async_copy(pki_hbm.at[blk0 + b], pbufs[q], esems[q])
        pltpu.async_copy(pkf_hbm.at[blk0 + b], fbufs[q], esems[q])

    def wait_edge(b, q):
        pltpu.make_async_copy(pki_hbm.at[blk0 + b], pbufs[q],
                              esems[q]).wait()
        pltpu.make_async_copy(pkf_hbm.at[blk0 + b], fbufs[q],
                              esems[q]).wait()

    def fire_gathers(q):
        pltpu.async_copy(x_hbm.at[pbufs[q].at[1]], rbh[q], gsems[q])
        pltpu.async_copy(nt_hbm.at[pbufs[q].at[0]], ntds[q], gsems[q])

    def wait_gathers(q):
        pltpu.make_async_copy(x_hbm.at[pbufs[q].at[1]], rbh[q],
                              gsems[q]).wait()
        pltpu.make_async_copy(nt_hbm.at[pbufs[q].at[0]], ntds[q],
                              gsems[q]).wait()

    def fire_scatter(q):
        pltpu.async_copy(rbh[q], agg_sh.at[dbufs[q]], ssems[q], add=True)

    def wait_scatter(q):
        pltpu.make_async_copy(rbh[q], agg_sh.at[dbufs[q]], ssems[q]).wait()

    def compute(q):
        rq = rbh[q]

        def grp(g, carry):
            nt = ntds[q][pl.ds(g * L, L)]
            etv = fbufs[q][0, pl.ds(g * L, L)]
            ewv = fbufs[q][1, pl.ds(g * L, L)]
            cvec = ewv * jnp.exp(-jnp.abs(nt - etv))
            dbufs[q][pl.ds(g * L, L)] = pbufs[q][0, pl.ds(g * L, L)]
            for li in range(L):
                cf = cvec[li]
                e = g * L + li
                for j in range(D // L):
                    rq[e, pl.ds(j * L, L)] = rq[e, pl.ds(j * L, L)] * cf
            return carry

        lax.fori_loop(0, KB // L, grp, 0)

    edge_copy(0, 0)
    edge_copy(1, 1)
    wait_edge(0, 0)
    fire_gathers(0)

    def step(b, q):
        pa = (q + 1) % NBUF
        pp = (q + 2) % NBUF

        @pl.when(b >= 2)
        def _retire():
            wait_scatter(pa)

        @pl.when(b + 1 < NB)
        def _advance():
            wait_edge(b + 1, pa)
            fire_gathers(pa)

        wait_gathers(q)
        compute(q)
        fire_scatter(q)

        @pl.when(b + 2 < NB)
        def _prefetch():
            edge_copy(b + 2, pp)

    def superstep(i, carry):
        for k in range(NBUF):
            step(i * NBUF + k, k)
        return carry

    lax.fori_loop(0, NB // NBUF, superstep, 0)
    wait_scatter((NB - 2) % NBUF)
    wait_scatter((NB - 1) % NBUF)

    plsc.subcore_barrier()
    pltpu.sync_copy(agg_sh.at[pl.ds(row0, RPT)],
                    out_hbm.at[c, pl.ds(row0, RPT)])

    @pl.when(s == NS - 1)
    def _emit_tail():
        pltpu.sync_copy(agg_sh.at[pl.ds(NS * RPT, RREM)],
                        out_hbm.at[c, pl.ds(NS * RPT, RREM)])


_sc_aggregate = pl.kernel(
    _sc_body,
    out_type=jax.ShapeDtypeStruct((NC, N, D), jnp.float32),
    mesh=plsc.VectorSubcoreMesh(
        core_axis_name="c", subcore_axis_name="s", num_cores=NC,
        num_subcores=NS,
    ),
    scratch_types=(
        [pltpu.VMEM_SHARED((N, D), jnp.float32)]
        + [pltpu.VMEM((2, KB), jnp.int32) for _ in range(NBUF)]
        + [pltpu.VMEM((2, KB), jnp.float32) for _ in range(NBUF)]
        + [pltpu.VMEM((KB, D), jnp.float32) for _ in range(NBUF)]
        + [pltpu.VMEM((KB,), jnp.float32) for _ in range(NBUF)]
        + [pltpu.VMEM((KB,), jnp.int32) for _ in range(NBUF)]
        + [pltpu.SemaphoreType.DMA for _ in range(3 * NBUF)]
    ),
)


def _dense_body(p_ref, x_ref, wg_ref, w1_ref, b1_ref, g_ref, be_ref, w2_ref,
                b2_ref, o_ref):
    agg = p_ref[0] + p_ref[1] - x_ref[...]
    ne = jnp.maximum(jnp.dot(agg, wg_ref[...],
                             preferred_element_type=jnp.float32), 0.0)
    h = jnp.dot(ne, w1_ref[...], preferred_element_type=jnp.float32)
    h = jnp.maximum(h + b1_ref[...], 0.0)
    mean = jnp.mean(h, axis=0, keepdims=True)
    var = jnp.mean((h - mean) * (h - mean), axis=0, keepdims=True)
    hn = (h - mean) / jnp.sqrt(var + 1e-5) * g_ref[...] + be_ref[...]
    out = jnp.dot(hn, w2_ref[...], preferred_element_type=jnp.float32)
    o_ref[...] = jax.nn.sigmoid(out + b2_ref[...])


_dense_call = pl.pallas_call(
    _dense_body,
    out_shape=jax.ShapeDtypeStruct((N, H2), jnp.float32),
)


def _pack_edges(edge_index, edge_time, edge_weight):
    pad = EPW_P - EPW
    dst = edge_index[1].reshape(NW, EPW)
    src = edge_index[0].reshape(NW, EPW)
    et = edge_time.reshape(NW, EPW)
    ew = edge_weight.reshape(NW, EPW)
    cfg = [(0, 0), (0, pad)]
    dst, src, et, ew = [jnp.pad(a, cfg).reshape(NW, NB, KB)
                        for a in (dst, src, et, ew)]
    pki = jnp.stack([dst, src], axis=2).reshape(NW * NB, 2, KB)
    pkf = jnp.stack([et, ew], axis=2).reshape(NW * NB, 2, KB)
    return pki, pkf


def kernel(x, edge_index, edge_time, node_time, edge_weight,
           W_g, W1, b1, gamma, beta, W2, b2):
    pki, pkf = _pack_edges(edge_index, edge_time, edge_weight)
    partials = _sc_aggregate(x, pki, pkf, node_time)
    return _dense_call(partials, x, W_g, W1, b1.reshape(1, H1),
                       gamma.reshape(1, H1), beta.reshape(1, H1), W2,
                       b2.reshape(1, H2))


# speedup vs baseline: 1.1592x; 1.0953x over previous
import jax
import jax.numpy as jnp
from jax import lax
from jax.experimental import pallas as pl
from jax.experimental.pallas import tpu as pltpu
from jax.experimental.pallas import tpu_sc as plsc

N, E, D = 10000, 320000, 128
H1, H2 = 128, 64
NC, NS, L = 2, 16, 16
NW = NC * NS
EPW = E // NW
KB = 112
NB = 90
EPW_P = NB * KB
NBUF = 3
GSZ = 3 * KB
NGRP = NB // 3
RPT = 624
RREM = N - NS * RPT


def _sc_body(x_hbm, src_hbm, dst_hbm, et_hbm, ew_hbm, nt_hbm, out_hbm,
             agg_sh,
             sb0, sb1, sb2, tb0, tb1, tb2, eb0, eb1, eb2, wb0, wb1, wb2,
             rb0, rb1, rb2, nd0, nd1, nd2, db0, db1, db2,
             ec0, ec1, ec2, gs0, gs1, gs2, ss0, ss1, ss2):
    sbufs = (sb0, sb1, sb2)
    tbufs = (tb0, tb1, tb2)
    ebufs = (eb0, eb1, eb2)
    wbufs = (wb0, wb1, wb2)
    rbh = (rb0, rb1, rb2)
    ntds = (nd0, nd1, nd2)
    dbufs = (db0, db1, db2)
    gcsems = (ec0, ec1, ec2)
    gsems = (gs0, gs1, gs2)
    ssems = (ss0, ss1, ss2)

    c = lax.axis_index("c")
    s = lax.axis_index("s")
    wid = c * NS + s

    row0 = s * RPT
    pltpu.sync_copy(x_hbm.at[pl.ds(row0, RPT)], agg_sh.at[pl.ds(row0, RPT)])

    @pl.when(s == NS - 1)
    def _init_tail():
        pltpu.sync_copy(x_hbm.at[pl.ds(NS * RPT, RREM)],
                        agg_sh.at[pl.ds(NS * RPT, RREM)])

    plsc.subcore_barrier()

    ebase = wid * EPW_P

    def group_copy(g, r):
        off = ebase + g * GSZ
        pltpu.async_copy(src_hbm.at[pl.ds(off, GSZ)], sbufs[r], gcsems[r])
        pltpu.async_copy(dst_hbm.at[pl.ds(off, GSZ)], tbufs[r], gcsems[r])
        pltpu.async_copy(et_hbm.at[pl.ds(off, GSZ)], ebufs[r], gcsems[r])
        pltpu.async_copy(ew_hbm.at[pl.ds(off, GSZ)], wbufs[r], gcsems[r])

    def wait_group(g, r):
        off = ebase + g * GSZ
        pltpu.make_async_copy(src_hbm.at[pl.ds(off, GSZ)], sbufs[r],
                              gcsems[r]).wait()
        pltpu.make_async_copy(dst_hbm.at[pl.ds(off, GSZ)], tbufs[r],
                              gcsems[r]).wait()
        pltpu.make_async_copy(et_hbm.at[pl.ds(off, GSZ)], ebufs[r],
                              gcsems[r]).wait()
        pltpu.make_async_copy(ew_hbm.at[pl.ds(off, GSZ)], wbufs[r],
                              gcsems[r]).wait()

    def fire_gathers(q, r, jj):
        pltpu.async_copy(x_hbm.at[sbufs[r].at[pl.ds(jj * KB, KB)]],
                         rbh[q], gsems[q])
        pltpu.async_copy(nt_hbm.at[tbufs[r].at[pl.ds(jj * KB, KB)]],
                         ntds[q], gsems[q])

    def wait_gathers(q, r, jj):
        pltpu.make_async_copy(x_hbm.at[sbufs[r].at[pl.ds(jj * KB, KB)]],
                              rbh[q], gsems[q]).wait()
        pltpu.make_async_copy(nt_hbm.at[tbufs[r].at[pl.ds(jj * KB, KB)]],
                              ntds[q], gsems[q]).wait()

    def fire_scatter(q):
        pltpu.async_copy(rbh[q], agg_sh.at[dbufs[q]], ssems[q], add=True)

    def wait_scatter(q):
        pltpu.make_async_copy(rbh[q], agg_sh.at[dbufs[q]], ssems[q]).wait()

    def compute(q, r, jj):
        rq = rbh[q]

        def grp(g, carry):
            nt = ntds[q][pl.ds(g * L, L)]
            etv = ebufs[r][pl.ds(jj * KB + g * L, L)]
            ewv = wbufs[r][pl.ds(jj * KB + g * L, L)]
            cvec = ewv * jnp.exp(-jnp.abs(nt - etv))
            dbufs[q][pl.ds(g * L, L)] = tbufs[r][pl.ds(jj * KB + g * L, L)]
            for li in range(L):
                cf = cvec[li]
                e = g * L + li
                for j in range(D // L):
                    rq[e, pl.ds(j * L, L)] = rq[e, pl.ds(j * L, L)] * cf
            return carry

        lax.fori_loop(0, KB // L, grp, 0)

    group_copy(0, 0)
    group_copy(1, 1)
    wait_group(0, 0)
    fire_gathers(0, 0, 0)

    def step(b, k):
        q = k % 3
        pa = (q + 1) % 3
        rc = (k // 3) % 3
        jc = k % 3
        rn = ((k + 1) // 3) % 3
        jn = (k + 1) % 3

        if k % 3 == 0:
            gnew = b // 3 + 2

            @pl.when(gnew < NGRP)
            def _stage_group():
                group_copy(gnew, (k // 3 + 2) % 3)

        @pl.when(b >= 2)
        def _retire():
            wait_scatter(pa)

        @pl.when(b + 1 < NB)
        def _advance():
            if jn == 0:
                wait_group((b + 1) // 3, rn)
            fire_gathers(pa, rn, jn)

        wait_gathers(q, rc, jc)
        compute(q, rc, jc)
        fire_scatter(q)

    def superstep(i, carry):
        for k in range(9):
            step(i * 9 + k, k)
        return carry

    lax.fori_loop(0, NB // 9, superstep, 0)
    wait_scatter((NB - 2) % 3)
    wait_scatter((NB - 1) % 3)

    plsc.subcore_barrier()
    pltpu.sync_copy(agg_sh.at[pl.ds(row0, RPT)],
                    out_hbm.at[c, pl.ds(row0, RPT)])

    @pl.when(s == NS - 1)
    def _emit_tail():
        pltpu.sync_copy(agg_sh.at[pl.ds(NS * RPT, RREM)],
                        out_hbm.at[c, pl.ds(NS * RPT, RREM)])


_sc_aggregate = pl.kernel(
    _sc_body,
    out_type=jax.ShapeDtypeStruct((NC, N, D), jnp.float32),
    mesh=plsc.VectorSubcoreMesh(
        core_axis_name="c", subcore_axis_name="s", num_cores=NC,
        num_subcores=NS,
    ),
    scratch_types=(
        [pltpu.VMEM_SHARED((N, D), jnp.float32)]
        + [pltpu.VMEM((GSZ,), jnp.int32) for _ in range(3)]
        + [pltpu.VMEM((GSZ,), jnp.int32) for _ in range(3)]
        + [pltpu.VMEM((GSZ,), jnp.float32) for _ in range(3)]
        + [pltpu.VMEM((GSZ,), jnp.float32) for _ in range(3)]
        + [pltpu.VMEM((KB, D), jnp.float32) for _ in range(3)]
        + [pltpu.VMEM((KB,), jnp.float32) for _ in range(3)]
        + [pltpu.VMEM((KB,), jnp.int32) for _ in range(3)]
        + [pltpu.SemaphoreType.DMA for _ in range(9)]
    ),
)


def _dense_body(p_ref, x_ref, wg_ref, w1_ref, b1_ref, g_ref, be_ref, w2_ref,
                b2_ref, o_ref):
    agg = p_ref[0] + p_ref[1] - x_ref[...]
    ne = jnp.maximum(jnp.dot(agg, wg_ref[...],
                             preferred_element_type=jnp.float32), 0.0)
    h = jnp.dot(ne, w1_ref[...], preferred_element_type=jnp.float32)
    h = jnp.maximum(h + b1_ref[...], 0.0)
    mean = jnp.mean(h, axis=0, keepdims=True)
    var = jnp.mean((h - mean) * (h - mean), axis=0, keepdims=True)
    hn = (h - mean) / jnp.sqrt(var + 1e-5) * g_ref[...] + be_ref[...]
    out = jnp.dot(hn, w2_ref[...], preferred_element_type=jnp.float32)
    o_ref[...] = jax.nn.sigmoid(out + b2_ref[...])


_dense_call = pl.pallas_call(
    _dense_body,
    out_shape=jax.ShapeDtypeStruct((N, H2), jnp.float32),
)


def _pad_edges(edge_index, edge_time, edge_weight):
    pad = EPW_P - EPW
    dst = edge_index[1].reshape(NW, EPW)
    src = edge_index[0].reshape(NW, EPW)
    et = edge_time.reshape(NW, EPW)
    ew = edge_weight.reshape(NW, EPW)
    cfg = [(0, 0), (0, pad)]
    return [jnp.pad(a, cfg).reshape(NW * EPW_P)
            for a in (src, dst, et, ew)]


def kernel(x, edge_index, edge_time, node_time, edge_weight,
           W_g, W1, b1, gamma, beta, W2, b2):
    srcp, dstp, etp, ewp = _pad_edges(edge_index, edge_time, edge_weight)
    partials = _sc_aggregate(x, srcp, dstp, etp, ewp, node_time)
    return _dense_call(partials, x, W_g, W1, b1.reshape(1, H1),
                       gamma.reshape(1, H1), beta.reshape(1, H1), W2,
                       b2.reshape(1, H2))
